# Initial kernel scaffold; baseline (speedup 1.0000x reference)
import jax
import jax.numpy as jnp
from jax.experimental import pallas as pl


def kernel(x, W, b):
    raise NotImplementedError("write your pallas kernel here")


# speedup vs baseline: 1.3124x; 1.3124x over previous
import functools

import jax
import jax.numpy as jnp
from jax.experimental import pallas as pl
from jax.experimental.pallas import tpu as pltpu

_TK = 2


def _act_body(x_ref, act_ref, idx_ref, *, nb, nc, in_ch, inv_count):
    i = pl.program_id(0)

    @pl.when(i == 0)
    def _init():
        for c in range(nc):
            act_ref[c] = 0.0

    xa = jnp.abs(x_ref[...])
    for c in range(nc):
        act_ref[c] += jnp.sum(xa[:, c * in_ch:(c + 1) * in_ch])

    @pl.when(i == nb - 1)
    def _finalize():
        a = [act_ref[c] * inv_count for c in range(nc)]
        for c in range(nc):
            act_ref[c] = a[c]
        best = a[0]
        bi = jnp.int32(0)
        for c in range(1, nc):
            cond = a[c] > best
            best = jnp.where(cond, a[c], best)
            bi = jnp.where(cond, jnp.int32(c), bi)
        idx_ref[0] = bi
        best2 = jnp.float32(-jnp.inf)
        bi2 = jnp.int32(0)
        for c in range(nc):
            v = jnp.where(jnp.int32(c) == bi, -jnp.inf, a[c])
            cond = v > best2
            best2 = jnp.where(cond, v, best2)
            bi2 = jnp.where(cond, jnp.int32(c), bi2)
        idx_ref[1] = bi2


def _mm_body(idx_ref, x_ref, w_ref, b_ref, o_ref):
    c = pl.program_id(0)
    sel = (c == idx_ref[0]) | (c == idx_ref[1])

    @pl.when(sel)
    def _compute():
        acc = jax.lax.dot_general(
            x_ref[...], w_ref[0],
            dimension_numbers=(((1,), (1,)), ((), ())),
            preferred_element_type=jnp.float32)
        o_ref[...] = acc + b_ref[0]

    @pl.when(jnp.logical_not(sel))
    def _zero():
        o_ref[...] = jnp.zeros_like(o_ref)


def kernel(x, W, b):
    n_tok, in_f = x.shape
    nc, out_ch, in_ch = W.shape
    out_f = nc * out_ch

    bn1 = 512
    nb1 = n_tok // bn1
    act, idx = pl.pallas_call(
        functools.partial(_act_body, nb=nb1, nc=nc, in_ch=in_ch,
                          inv_count=1.0 / (n_tok * in_ch)),
        grid=(nb1,),
        in_specs=[pl.BlockSpec((bn1, in_f), lambda i: (i, 0))],
        out_specs=[pl.BlockSpec(memory_space=pltpu.SMEM),
                   pl.BlockSpec(memory_space=pltpu.SMEM)],
        out_shape=[jax.ShapeDtypeStruct((nc,), jnp.float32),
                   jax.ShapeDtypeStruct((_TK,), jnp.int32)],
    )(x)

    bn2 = 512
    nb2 = n_tok // bn2

    def x_map(c, n, idx_ref):
        sel = (c == idx_ref[0]) | (c == idx_ref[1])
        return (jnp.where(sel, n, 0), jnp.where(sel, c, idx_ref[0]))

    def w_map(c, n, idx_ref):
        sel = (c == idx_ref[0]) | (c == idx_ref[1])
        return (jnp.where(sel, c, idx_ref[0]), 0, 0)

    def b_map(c, n, idx_ref):
        sel = (c == idx_ref[0]) | (c == idx_ref[1])
        return (jnp.where(sel, c, idx_ref[0]), 0, 0)

    out = pl.pallas_call(
        _mm_body,
        grid_spec=pltpu.PrefetchScalarGridSpec(
            num_scalar_prefetch=1,
            grid=(nc, nb2),
            in_specs=[
                pl.BlockSpec((bn2, in_ch), x_map),
                pl.BlockSpec((1, out_ch, in_ch), w_map),
                pl.BlockSpec((1, 1, out_ch), b_map),
            ],
            out_specs=pl.BlockSpec((bn2, out_ch), lambda c, n, idx_ref: (n, c)),
        ),
        out_shape=jax.ShapeDtypeStruct((n_tok, out_f), jnp.float32),
    )(idx, x, W, b.reshape(nc, 1, out_ch))

    return out, act


# speedup vs baseline: 1.7851x; 1.3601x over previous
import functools

import jax
import jax.numpy as jnp
from jax.experimental import pallas as pl
from jax.experimental.pallas import tpu as pltpu

_TK = 2


def _act_body(x_ref, act_ref, idx_ref, *, nb, nc, in_ch, inv_count):
    i = pl.program_id(0)

    @pl.when(i == 0)
    def _init():
        for c in range(nc):
            act_ref[c] = 0.0

    xa = jnp.abs(x_ref[...])
    for c in range(nc):
        act_ref[c] += jnp.sum(xa[:, c * in_ch:(c + 1) * in_ch])

    @pl.when(i == nb - 1)
    def _finalize():
        a = [act_ref[c] * inv_count for c in range(nc)]
        for c in range(nc):
            act_ref[c] = a[c]
        best = a[0]
        bi = jnp.int32(0)
        for c in range(1, nc):
            cond = a[c] > best
            best = jnp.where(cond, a[c], best)
            bi = jnp.where(cond, jnp.int32(c), bi)
        idx_ref[0] = bi
        best2 = jnp.float32(-jnp.inf)
        bi2 = jnp.int32(0)
        for c in range(nc):
            v = jnp.where(jnp.int32(c) == bi, -jnp.inf, a[c])
            cond = v > best2
            best2 = jnp.where(cond, v, best2)
            bi2 = jnp.where(cond, jnp.int32(c), bi2)
        idx_ref[1] = bi2


def _mm_body(idx_ref, x_ref, w_ref, b_ref, o_ref):
    c = pl.program_id(0)
    sel = (c == idx_ref[0]) | (c == idx_ref[1])

    @pl.when(sel)
    def _compute():
        acc = jax.lax.dot_general(
            x_ref[...], w_ref[0],
            dimension_numbers=(((1,), (1,)), ((), ())),
            preferred_element_type=jnp.float32)
        o_ref[...] = acc + b_ref[0]

    @pl.when(jnp.logical_not(sel))
    def _zero():
        o_ref[...] = jnp.zeros_like(o_ref)


def kernel(x, W, b):
    n_tok, in_f = x.shape
    nc, out_ch, in_ch = W.shape
    out_f = nc * out_ch

    bn1 = 1024
    nb1 = n_tok // bn1
    act, idx = pl.pallas_call(
        functools.partial(_act_body, nb=nb1, nc=nc, in_ch=in_ch,
                          inv_count=1.0 / (n_tok * in_ch)),
        grid=(nb1,),
        in_specs=[pl.BlockSpec((bn1, in_f), lambda i: (i, 0))],
        out_specs=[pl.BlockSpec(memory_space=pltpu.SMEM),
                   pl.BlockSpec(memory_space=pltpu.SMEM)],
        out_shape=[jax.ShapeDtypeStruct((nc,), jnp.float32),
                   jax.ShapeDtypeStruct((_TK,), jnp.int32)],
    )(x)

    bn2 = 1024
    nb2 = n_tok // bn2

    def x_map(c, n, idx_ref):
        sel = (c == idx_ref[0]) | (c == idx_ref[1])
        return (jnp.where(sel, n, 0), jnp.where(sel, c, idx_ref[0]))

    def w_map(c, n, idx_ref):
        sel = (c == idx_ref[0]) | (c == idx_ref[1])
        return (jnp.where(sel, c, idx_ref[0]), 0, 0)

    def b_map(c, n, idx_ref):
        sel = (c == idx_ref[0]) | (c == idx_ref[1])
        return (jnp.where(sel, c, idx_ref[0]), 0, 0)

    out = pl.pallas_call(
        _mm_body,
        grid_spec=pltpu.PrefetchScalarGridSpec(
            num_scalar_prefetch=1,
            grid=(nc, nb2),
            in_specs=[
                pl.BlockSpec((bn2, in_ch), x_map),
                pl.BlockSpec((1, out_ch, in_ch), w_map),
                pl.BlockSpec((1, 1, out_ch), b_map),
            ],
            out_specs=pl.BlockSpec((bn2, out_ch), lambda c, n, idx_ref: (n, c)),
        ),
        out_shape=jax.ShapeDtypeStruct((n_tok, out_f), jnp.float32),
    )(idx, x, W, b.reshape(nc, 1, out_ch))

    return out, act


# speedup vs baseline: 2.1378x; 1.1976x over previous
import functools

import jax
import jax.numpy as jnp
from jax.experimental import pallas as pl
from jax.experimental.pallas import tpu as pltpu

_TK = 2


def _act_body(x_ref, act_ref, idx_ref, *, nb, nc, in_ch, inv_count):
    i = pl.program_id(0)

    @pl.when(i == 0)
    def _init():
        for c in range(nc):
            act_ref[c] = 0.0

    xa = jnp.abs(x_ref[...])
    for c in range(nc):
        act_ref[c] += jnp.sum(xa[:, c * in_ch:(c + 1) * in_ch])

    @pl.when(i == nb - 1)
    def _finalize():
        a = [act_ref[c] * inv_count for c in range(nc)]
        for c in range(nc):
            act_ref[c] = a[c]
        best = a[0]
        bi = jnp.int32(0)
        for c in range(1, nc):
            cond = a[c] > best
            best = jnp.where(cond, a[c], best)
            bi = jnp.where(cond, jnp.int32(c), bi)
        idx_ref[0] = bi
        best2 = jnp.float32(-jnp.inf)
        bi2 = jnp.int32(0)
        for c in range(nc):
            v = jnp.where(jnp.int32(c) == bi, -jnp.inf, a[c])
            cond = v > best2
            best2 = jnp.where(cond, v, best2)
            bi2 = jnp.where(cond, jnp.int32(c), bi2)
        idx_ref[1] = bi2


def _mm_body(idx_ref, x_ref, w_ref, b_ref, o_ref):
    c = pl.program_id(0)
    sel = (c == idx_ref[0]) | (c == idx_ref[1])

    @pl.when(sel)
    def _compute():
        acc = jax.lax.dot_general(
            x_ref[...], w_ref[0],
            dimension_numbers=(((1,), (1,)), ((), ())),
            preferred_element_type=jnp.float32)
        o_ref[...] = acc + b_ref[0]

    @pl.when(jnp.logical_not(sel))
    def _zero():
        o_ref[...] = jnp.zeros_like(o_ref)


def kernel(x, W, b):
    n_tok, in_f = x.shape
    nc, out_ch, in_ch = W.shape
    out_f = nc * out_ch

    bn1 = 2048
    nb1 = n_tok // bn1
    act, idx = pl.pallas_call(
        functools.partial(_act_body, nb=nb1, nc=nc, in_ch=in_ch,
                          inv_count=1.0 / (n_tok * in_ch)),
        grid=(nb1,),
        in_specs=[pl.BlockSpec((bn1, in_f), lambda i: (i, 0))],
        out_specs=[pl.BlockSpec(memory_space=pltpu.SMEM),
                   pl.BlockSpec(memory_space=pltpu.SMEM)],
        out_shape=[jax.ShapeDtypeStruct((nc,), jnp.float32),
                   jax.ShapeDtypeStruct((_TK,), jnp.int32)],
    )(x)

    bn2 = 2048
    nb2 = n_tok // bn2

    def x_map(c, n, idx_ref):
        sel = (c == idx_ref[0]) | (c == idx_ref[1])
        return (jnp.where(sel, n, 0), jnp.where(sel, c, idx_ref[0]))

    def w_map(c, n, idx_ref):
        sel = (c == idx_ref[0]) | (c == idx_ref[1])
        return (jnp.where(sel, c, idx_ref[0]), 0, 0)

    def b_map(c, n, idx_ref):
        sel = (c == idx_ref[0]) | (c == idx_ref[1])
        return (jnp.where(sel, c, idx_ref[0]), 0, 0)

    out = pl.pallas_call(
        _mm_body,
        grid_spec=pltpu.PrefetchScalarGridSpec(
            num_scalar_prefetch=1,
            grid=(nc, nb2),
            in_specs=[
                pl.BlockSpec((bn2, in_ch), x_map),
                pl.BlockSpec((1, out_ch, in_ch), w_map),
                pl.BlockSpec((1, 1, out_ch), b_map),
            ],
            out_specs=pl.BlockSpec((bn2, out_ch), lambda c, n, idx_ref: (n, c)),
        ),
        out_shape=jax.ShapeDtypeStruct((n_tok, out_f), jnp.float32),
    )(idx, x, W, b.reshape(nc, 1, out_ch))

    return out, act


# speedup vs baseline: 2.2609x; 1.0576x over previous
import functools

import jax
import jax.numpy as jnp
from jax.experimental import pallas as pl
from jax.experimental.pallas import tpu as pltpu

_TK = 2


def _act_body(x_ref, act_ref, idx_ref, *, nb, nc, in_ch, inv_count):
    i = pl.program_id(0)

    @pl.when(i == 0)
    def _init():
        for c in range(nc):
            act_ref[c] = 0.0

    xa = jnp.abs(x_ref[...])
    for c in range(nc):
        act_ref[c] += jnp.sum(xa[:, c * in_ch:(c + 1) * in_ch])

    @pl.when(i == nb - 1)
    def _finalize():
        a = [act_ref[c] * inv_count for c in range(nc)]
        for c in range(nc):
            act_ref[c] = a[c]
        best = a[0]
        bi = jnp.int32(0)
        for c in range(1, nc):
            cond = a[c] > best
            best = jnp.where(cond, a[c], best)
            bi = jnp.where(cond, jnp.int32(c), bi)
        idx_ref[0] = bi
        best2 = jnp.float32(-jnp.inf)
        bi2 = jnp.int32(0)
        for c in range(nc):
            v = jnp.where(jnp.int32(c) == bi, -jnp.inf, a[c])
            cond = v > best2
            best2 = jnp.where(cond, v, best2)
            bi2 = jnp.where(cond, jnp.int32(c), bi2)
        idx_ref[1] = bi2


def _mm_body(idx_ref, x_ref, w_ref, b_ref, o_ref):
    c = pl.program_id(0)
    sel = (c == idx_ref[0]) | (c == idx_ref[1])

    @pl.when(sel)
    def _compute():
        acc = jax.lax.dot_general(
            x_ref[...], w_ref[0],
            dimension_numbers=(((1,), (1,)), ((), ())),
            preferred_element_type=jnp.float32)
        o_ref[...] = acc + b_ref[0]

    @pl.when(jnp.logical_not(sel))
    def _zero():
        o_ref[...] = jnp.zeros_like(o_ref)


def kernel(x, W, b):
    n_tok, in_f = x.shape
    nc, out_ch, in_ch = W.shape
    out_f = nc * out_ch

    bn1 = 2048
    nb1 = n_tok // bn1
    act, idx = pl.pallas_call(
        functools.partial(_act_body, nb=nb1, nc=nc, in_ch=in_ch,
                          inv_count=1.0 / (n_tok * in_ch)),
        grid=(nb1,),
        in_specs=[pl.BlockSpec((bn1, in_f), lambda i: (i, 0))],
        out_specs=[pl.BlockSpec(memory_space=pltpu.SMEM),
                   pl.BlockSpec(memory_space=pltpu.SMEM)],
        out_shape=[jax.ShapeDtypeStruct((nc,), jnp.float32),
                   jax.ShapeDtypeStruct((_TK,), jnp.int32)],
    )(x)

    bn2 = 4096
    nb2 = n_tok // bn2

    def x_map(c, n, idx_ref):
        sel = (c == idx_ref[0]) | (c == idx_ref[1])
        return (jnp.where(sel, n, 0), jnp.where(sel, c, idx_ref[0]))

    def w_map(c, n, idx_ref):
        sel = (c == idx_ref[0]) | (c == idx_ref[1])
        return (jnp.where(sel, c, idx_ref[0]), 0, 0)

    def b_map(c, n, idx_ref):
        sel = (c == idx_ref[0]) | (c == idx_ref[1])
        return (jnp.where(sel, c, idx_ref[0]), 0, 0)

    out = pl.pallas_call(
        _mm_body,
        grid_spec=pltpu.PrefetchScalarGridSpec(
            num_scalar_prefetch=1,
            grid=(nc, nb2),
            in_specs=[
                pl.BlockSpec((bn2, in_ch), x_map),
                pl.BlockSpec((1, out_ch, in_ch), w_map),
                pl.BlockSpec((1, 1, out_ch), b_map),
            ],
            out_specs=pl.BlockSpec((bn2, out_ch), lambda c, n, idx_ref: (n, c)),
        ),
        out_shape=jax.ShapeDtypeStruct((n_tok, out_f), jnp.float32),
    )(idx, x, W, b.reshape(nc, 1, out_ch))

    return out, act


# speedup vs baseline: 2.3321x; 1.0315x over previous
import functools

import jax
import jax.numpy as jnp
from jax.experimental import pallas as pl
from jax.experimental.pallas import tpu as pltpu

_TK = 2


def _act_body(x_ref, act_ref, idx_ref, *, nb, nc, in_ch, inv_count):
    i = pl.program_id(0)

    @pl.when(i == 0)
    def _init():
        for c in range(nc):
            act_ref[c] = 0.0

    xa = jnp.abs(x_ref[...])
    for c in range(nc):
        act_ref[c] += jnp.sum(xa[:, c * in_ch:(c + 1) * in_ch])

    @pl.when(i == nb - 1)
    def _finalize():
        a = [act_ref[c] * inv_count for c in range(nc)]
        for c in range(nc):
            act_ref[c] = a[c]
        best = a[0]
        bi = jnp.int32(0)
        for c in range(1, nc):
            cond = a[c] > best
            best = jnp.where(cond, a[c], best)
            bi = jnp.where(cond, jnp.int32(c), bi)
        idx_ref[0] = bi
        best2 = jnp.float32(-jnp.inf)
        bi2 = jnp.int32(0)
        for c in range(nc):
            v = jnp.where(jnp.int32(c) == bi, -jnp.inf, a[c])
            cond = v > best2
            best2 = jnp.where(cond, v, best2)
            bi2 = jnp.where(cond, jnp.int32(c), bi2)
        idx_ref[1] = bi2


def _mm_body(idx_ref, x_ref, w_ref, b_ref, o_ref):
    c = pl.program_id(0)
    sel = (c == idx_ref[0]) | (c == idx_ref[1])

    @pl.when(sel)
    def _compute():
        acc = jax.lax.dot_general(
            x_ref[...], w_ref[0],
            dimension_numbers=(((1,), (1,)), ((), ())),
            preferred_element_type=jnp.float32)
        o_ref[...] = acc + b_ref[0]

    @pl.when(jnp.logical_not(sel))
    def _zero():
        o_ref[...] = jnp.zeros_like(o_ref)


def kernel(x, W, b):
    n_tok, in_f = x.shape
    nc, out_ch, in_ch = W.shape
    out_f = nc * out_ch

    bn1 = 2048
    nb1 = n_tok // bn1
    act, idx = pl.pallas_call(
        functools.partial(_act_body, nb=nb1, nc=nc, in_ch=in_ch,
                          inv_count=1.0 / (n_tok * in_ch)),
        grid=(nb1,),
        in_specs=[pl.BlockSpec((bn1, in_f), lambda i: (i, 0))],
        out_specs=[pl.BlockSpec(memory_space=pltpu.SMEM),
                   pl.BlockSpec(memory_space=pltpu.SMEM)],
        out_shape=[jax.ShapeDtypeStruct((nc,), jnp.float32),
                   jax.ShapeDtypeStruct((_TK,), jnp.int32)],
    )(x)

    bn2 = 8192
    nb2 = n_tok // bn2

    def x_map(c, n, idx_ref):
        sel = (c == idx_ref[0]) | (c == idx_ref[1])
        return (jnp.where(sel, n, 0), jnp.where(sel, c, idx_ref[0]))

    def w_map(c, n, idx_ref):
        sel = (c == idx_ref[0]) | (c == idx_ref[1])
        return (jnp.where(sel, c, idx_ref[0]), 0, 0)

    def b_map(c, n, idx_ref):
        sel = (c == idx_ref[0]) | (c == idx_ref[1])
        return (jnp.where(sel, c, idx_ref[0]), 0, 0)

    out = pl.pallas_call(
        _mm_body,
        grid_spec=pltpu.PrefetchScalarGridSpec(
            num_scalar_prefetch=1,
            grid=(nc, nb2),
            in_specs=[
                pl.BlockSpec((bn2, in_ch), x_map),
                pl.BlockSpec((1, out_ch, in_ch), w_map),
                pl.BlockSpec((1, 1, out_ch), b_map),
            ],
            out_specs=pl.BlockSpec((bn2, out_ch), lambda c, n, idx_ref: (n, c)),
        ),
        out_shape=jax.ShapeDtypeStruct((n_tok, out_f), jnp.float32),
    )(idx, x, W, b.reshape(nc, 1, out_ch))

    return out, act
